# Initial kernel scaffold; baseline (speedup 1.0000x reference)
import jax
import jax.numpy as jnp
from jax.experimental import pallas as pl


def kernel(x, edge_index, W1, b1, W2, b2):
    raise NotImplementedError("write your pallas kernel here")


# speedup vs baseline: 4.0714x; 4.0714x over previous
import functools

import jax
import jax.numpy as jnp
from jax import lax
from jax.experimental import pallas as pl
from jax.experimental.pallas import tpu as pltpu
from jax.experimental.pallas import tpu_sc as plsc

N, E, D = 10000, 160000, 256
HALF = D // 2
NTILES = 16
NP = 10112
RPT = NP // NTILES
EPT = E // NTILES
B = 128
NB = 80
NCHUNK = 2
CB = NB // NCHUNK

_mesh = plsc.VectorSubcoreMesh(core_axis_name="c", subcore_axis_name="s")


@functools.partial(
    pl.kernel,
    out_type=jax.ShapeDtypeStruct((2 * NP, HALF), jnp.float32),
    mesh=_mesh,
    scratch_types=[
        pltpu.VMEM((CB, B), jnp.int32),
        pltpu.VMEM((CB, B), jnp.int32),
        pltpu.VMEM((B, HALF), jnp.float32),
        pltpu.VMEM((B, HALF), jnp.float32),
        pltpu.VMEM_SHARED((NP, HALF), jnp.float32),
        pltpu.SemaphoreType.DMA,
        pltpu.SemaphoreType.DMA,
    ],
)
def _aggregate(xcat, srcg, dstg, out, src_v, dst_v, buf0, buf1, acc, sem0, sem1):
    c = lax.axis_index("c")
    s = lax.axis_index("s")
    r0 = pl.multiple_of(s * RPT, 8)
    base = pl.multiple_of(c * NP, 8)

    off = 0
    for sz in (128, 128, 128, 128, RPT - 512):
        pltpu.sync_copy(xcat.at[pl.ds(base + r0 + off, sz)],
                        acc.at[pl.ds(r0 + off, sz)])
        off += sz

    plsc.subcore_barrier()

    bufs = ((buf0, sem0), (buf1, sem1))

    def g_start(k, buf, sem):
        pltpu.async_copy(xcat.at[src_v.at[k]], buf, sem)

    def g_wait(k, buf, sem):
        pltpu.make_async_copy(xcat.at[src_v.at[k]], buf, sem).wait()

    for cc in range(NCHUNK):
        pltpu.sync_copy(srcg.at[c * NTILES + s].at[pl.ds(cc * CB, CB)], src_v)
        pltpu.sync_copy(dstg.at[s].at[pl.ds(cc * CB, CB)], dst_v)

        g_start(0, buf0, sem0)
        g_start(1, buf1, sem1)

        @pl.loop(0, CB // 2)
        def _pair(gp):
            for b in range(2):
                buf, sem = bufs[b]
                k = gp * 2 + b
                g_wait(k, buf, sem)
                pltpu.sync_copy(buf, acc.at[dst_v.at[k]], add=True)
                nk = k + 2

                @pl.when(nk < CB)
                def _():
                    g_start(nk, buf, sem)

    plsc.subcore_barrier()
    pltpu.sync_copy(acc.at[pl.ds(r0, RPT)], out.at[pl.ds(base + r0, RPT)])


def _mlp_body(s_ref, x_ref, w1_ref, b1_ref, w2_ref, b2_ref, o_ref):
    dn = (((1,), (0,)), ((), ()))
    h = jnp.concatenate([s_ref[0], s_ref[1]], axis=1)
    z = lax.dot_general(h, w1_ref[...], dn,
                        precision=lax.Precision.HIGHEST,
                        preferred_element_type=jnp.float32) + b1_ref[...]
    sp = jnp.maximum(z, 0.0) + jnp.log1p(jnp.exp(-jnp.abs(z)))
    h1 = z * jnp.tanh(sp)
    z2 = lax.dot_general(h1, w2_ref[...], dn,
                         precision=lax.Precision.HIGHEST,
                         preferred_element_type=jnp.float32) + b2_ref[...]
    o_ref[...] = x_ref[...] + z2


_BM = 1000


def kernel(x, edge_index, W1, b1, W2, b2):
    src = edge_index[0]
    dst = edge_index[1]

    xp = jnp.pad(x, ((0, NP - N), (0, 0)))
    xcat = xp.reshape(NP, 2, HALF).transpose(1, 0, 2).reshape(2 * NP, HALF)

    srcp = jnp.pad(src.reshape(NTILES, EPT), ((0, 0), (0, NB * B - EPT)))
    srcg = (srcp[None] + (jnp.arange(2, dtype=jnp.int32) * NP)[:, None, None])
    srcg = srcg.reshape(2 * NTILES, NB, B)
    dstg = jnp.pad(dst.reshape(NTILES, EPT), ((0, 0), (0, NB * B - EPT)),
                   constant_values=N).reshape(NTILES, NB, B)

    s3 = _aggregate(xcat, srcg, dstg).reshape(2, NP, HALF)

    return pl.pallas_call(
        _mlp_body,
        grid=(N // _BM,),
        in_specs=[
            pl.BlockSpec((2, _BM, HALF), lambda i: (0, i, 0)),
            pl.BlockSpec((_BM, D), lambda i: (i, 0)),
            pl.BlockSpec((D, D), lambda i: (0, 0)),
            pl.BlockSpec((1, D), lambda i: (0, 0)),
            pl.BlockSpec((D, D), lambda i: (0, 0)),
            pl.BlockSpec((1, D), lambda i: (0, 0)),
        ],
        out_specs=pl.BlockSpec((_BM, D), lambda i: (i, 0)),
        out_shape=jax.ShapeDtypeStruct((N, D), jnp.float32),
    )(s3, x, W1, b1.reshape(1, D), W2, b2.reshape(1, D))
